# Initial kernel scaffold; baseline (speedup 1.0000x reference)
import jax
import jax.numpy as jnp
from jax.experimental import pallas as pl


def kernel(h, node_embeddings, pos_edge_index, neg_edge_index, Wq_pos, bq_pos, Wk_pos, bk_pos, u_pos, Wq_neg, bq_neg, Wk_neg, bk_neg, u_neg, W_pos, b_pos, W_neg, b_neg, W_self, b_self):
    raise NotImplementedError("write your pallas kernel here")


# speedup vs baseline: 1.7499x; 1.7499x over previous
import functools

import jax
import jax.numpy as jnp
import numpy as np
from jax import lax
from jax.experimental import pallas as pl
from jax.experimental.pallas import tpu as pltpu
from jax.experimental.pallas import tpu_sc as plsc

N = 10000
E = 320000
IN_DIM = 128
EMB_DIM = 64
HID_DIM = 128
DROW = 208
BETA_POS = 0.1
BETA_NEG = -0.1
SCALE = float(np.sqrt(IN_DIM))

NC = 2
NS = 16
LANES = 16

EPT = E // NS
EB = 32
NBLK = EPT // EB
NGRP = EB // LANES
RSTRIDE = 624
RSPAN = 640

F32 = jnp.float32
I32 = jnp.int32


def _tables_body(h_ref, emb_ref, wqt_ref, bq_ref, wkt_ref, bk_ref, wt_ref,
                 b_ref, u_ref, u_out_ref, v_out_ref, hp_out_ref):
    x = h_ref[...]
    emb = emb_ref[...]
    wqt = wqt_ref[0]
    wkt = wkt_ref[0]
    wt = wt_ref[0]
    u = u_ref[0]
    q = jnp.dot(x, wqt, preferred_element_type=F32) + bq_ref[0]
    k = jnp.dot(x, wkt, preferred_element_type=F32) + bk_ref[0]
    p = jnp.sum(emb * u, axis=1, keepdims=True)
    pad = jnp.zeros((x.shape[0], DROW - IN_DIM - EMB_DIM - 1), dtype=F32)
    u_out_ref[...] = jnp.concatenate([q, emb, p, pad], axis=1)
    v_out_ref[...] = jnp.concatenate([k, emb, p, pad], axis=1)
    hp_out_ref[...] = jnp.dot(x, wt, preferred_element_type=F32) + b_ref[0]


def _build_tables(h, emb, wqt_s, bq_s, wkt_s, bk_s, wt_s, b_s, u_s):
    RB = 1000
    grid = (2, N // RB)
    return pl.pallas_call(
        _tables_body,
        grid=grid,
        in_specs=[
            pl.BlockSpec((RB, IN_DIM), lambda t, i: (i, 0)),
            pl.BlockSpec((RB, EMB_DIM), lambda t, i: (i, 0)),
            pl.BlockSpec((1, IN_DIM, IN_DIM), lambda t, i: (t, 0, 0)),
            pl.BlockSpec((1, 1, IN_DIM), lambda t, i: (t, 0, 0)),
            pl.BlockSpec((1, IN_DIM, IN_DIM), lambda t, i: (t, 0, 0)),
            pl.BlockSpec((1, 1, IN_DIM), lambda t, i: (t, 0, 0)),
            pl.BlockSpec((1, IN_DIM, HID_DIM), lambda t, i: (t, 0, 0)),
            pl.BlockSpec((1, 1, HID_DIM), lambda t, i: (t, 0, 0)),
            pl.BlockSpec((1, 1, EMB_DIM), lambda t, i: (t, 0, 0)),
        ],
        out_specs=[
            pl.BlockSpec((RB, DROW), lambda t, i: (t * (N // RB) + i, 0)),
            pl.BlockSpec((RB, DROW), lambda t, i: (t * (N // RB) + i, 0)),
            pl.BlockSpec((RB, HID_DIM), lambda t, i: (t * (N // RB) + i, 0)),
        ],
        out_shape=[
            jax.ShapeDtypeStruct((2 * N, DROW), F32),
            jax.ShapeDtypeStruct((2 * N, DROW), F32),
            jax.ShapeDtypeStruct((2 * N, HID_DIM), F32),
        ],
    )(h, emb, wqt_s, bq_s, wkt_s, bk_s, wt_s, b_s, u_s)


def _iota16():
    return lax.broadcasted_iota(I32, (LANES,), 0)


def _sc_body(edges_ref, u_tab_ref, v_tab_ref, hp_ref, zeros_ref, msgs_ref,
             urows, vrows, prow, scores_v, src_raw, dst_raw, src_b, dst_b,
             mb_v, splat_tmp, acc_sh, maxbuf, sumbuf, sem_u, sem_v, sem_p):
    c = lax.axis_index("c")
    s = lax.axis_index("s")
    c_n = c * N
    beta = jnp.where(c == 0, BETA_POS, BETA_NEG).astype(F32)
    edge_base = c * (2 * E) + s * EPT

    pltpu.sync_copy(zeros_ref.at[pl.ds(s * RSTRIDE, RSPAN), :],
                    acc_sh.at[pl.ds(s * RSTRIDE, RSPAN), :])

    iota = _iota16()

    def stage_indices(b, bias_dst):
        base = edge_base + b * EB
        pltpu.sync_copy(edges_ref.at[pl.ds(base, EB)], src_raw)
        pltpu.sync_copy(edges_ref.at[pl.ds(base + E, EB)], dst_raw)
        for k in range(EB // LANES):
            sl = pl.ds(k * LANES, LANES)
            src_b[sl] = src_raw[sl] + c_n
            if bias_dst:
                dst_b[sl] = dst_raw[sl] + c_n

    @pl.loop(0, NBLK)
    def _pass1(b):
        stage_indices(b, bias_dst=True)
        cp_u = pltpu.async_copy(u_tab_ref.at[src_b], urows, sem_u)
        cp_v = pltpu.async_copy(v_tab_ref.at[dst_b], vrows, sem_v)
        cp_u.wait()
        cp_v.wait()

        @pl.loop(0, NGRP)
        def _grp(g):
            rows = iota + g * LANES
            qk0 = jnp.zeros((LANES,), F32)
            qk1 = jnp.zeros((LANES,), F32)
            qk2 = jnp.zeros((LANES,), F32)
            qk3 = jnp.zeros((LANES,), F32)
            for f in range(0, IN_DIM, 4):
                for j in range(4):
                    col = jnp.full((LANES,), f + j, I32)
                    qv = plsc.load_gather(urows, [rows, col])
                    kv = plsc.load_gather(vrows, [rows, col])
                    if j == 0:
                        qk0 = qk0 + qv * kv
                    elif j == 1:
                        qk1 = qk1 + qv * kv
                    elif j == 2:
                        qk2 = qk2 + qv * kv
                    else:
                        qk3 = qk3 + qv * kv
            qk = (qk0 + qk1) + (qk2 + qk3)
            e20 = jnp.zeros((LANES,), F32)
            e21 = jnp.zeros((LANES,), F32)
            for f in range(0, EMB_DIM, 2):
                for j in range(2):
                    col = jnp.full((LANES,), IN_DIM + f + j, I32)
                    ev = plsc.load_gather(urows, [rows, col])
                    fv = plsc.load_gather(vrows, [rows, col])
                    dv = fv - ev
                    if j == 0:
                        e20 = e20 + dv * dv
                    else:
                        e21 = e21 + dv * dv
            e2 = e20 + e21
            colp = jnp.full((LANES,), IN_DIM + EMB_DIM, I32)
            pu = plsc.load_gather(urows, [rows, colp])
            pv = plsc.load_gather(vrows, [rows, colp])
            asym = pv - pu
            e2c = jnp.maximum(e2, 1e-30)
            yi = jnp.full((LANES,), 0x5F3759DF, I32) - (
                plsc.bitcast(e2c, I32) >> 1)
            y = plsc.bitcast(yi, F32)
            half = 0.5 * e2c
            for _ in range(3):
                y = y * (1.5 - half * y * y)
            euclid = e2c * y
            d = euclid + beta * asym
            sc = jnp.exp(-d) * qk / SCALE
            scores_v[pl.ds(b * EB + g * LANES, LANES)] = sc

    neg_inf = jnp.full((LANES,), -3.0e38, F32)

    @pl.loop(0, EPT // LANES, init_carry=neg_inf)
    def _lmax(i, mv):
        return jnp.maximum(mv, scores_v[pl.ds(i * LANES, LANES)])

    lmax = jnp.max(_lmax)
    splat_tmp[0, :] = jnp.full((LANES,), lmax, F32)
    pltpu.sync_copy(splat_tmp, maxbuf.at[pl.ds(s, 1), :])
    plsc.subcore_barrier()
    pltpu.sync_copy(maxbuf, mb_v)
    gmax = neg_inf
    for r in range(NS):
        gmax = jnp.maximum(gmax, mb_v[r, :])

    zero16 = jnp.zeros((LANES,), F32)

    @pl.loop(0, EPT // LANES, init_carry=zero16)
    def _lsum(i, sv):
        return sv + jnp.exp(scores_v[pl.ds(i * LANES, LANES)] - gmax)

    lsum = jnp.sum(_lsum)
    splat_tmp[0, :] = jnp.full((LANES,), lsum, F32)
    pltpu.sync_copy(splat_tmp, sumbuf.at[pl.ds(s, 1), :])
    plsc.subcore_barrier()
    pltpu.sync_copy(sumbuf, mb_v)
    gsum = zero16
    for r in range(NS):
        gsum = gsum + mb_v[r, :]
    inv_sum = 1.0 / gsum

    @pl.loop(0, EPT // LANES)
    def _att(i):
        sl = pl.ds(i * LANES, LANES)
        scores_v[sl] = jnp.exp(scores_v[sl] - gmax) * inv_sum

    @pl.loop(0, NBLK)
    def _pass2(b):
        stage_indices(b, bias_dst=False)
        cp_p = pltpu.async_copy(hp_ref.at[src_b], prow, sem_p)
        cp_p.wait()

        @pl.loop(0, EB)
        def _edge(e):
            att = plsc.load_gather(scores_v, [jnp.full((LANES,), b * EB + e, I32)])
            row = jnp.full((LANES,), e, I32)
            for ch in range(HID_DIM // LANES):
                col = iota + ch * LANES
                v = plsc.load_gather(prow, [row, col])
                plsc.store_scatter(prow, [row, col], v * att)

        pltpu.sync_copy(prow, acc_sh.at[dst_raw], add=True)

    plsc.subcore_barrier()
    pltpu.sync_copy(acc_sh.at[pl.ds(s * RSTRIDE, RSPAN), :],
                    msgs_ref.at[pl.ds(c * N + s * RSTRIDE, RSPAN), :])


def _sc_stage(edges_flat, u_tab, v_tab, hp_tab, zeros):
    mesh = plsc.VectorSubcoreMesh(core_axis_name="c", subcore_axis_name="s",
                                  num_cores=NC, num_subcores=NS)
    f = pl.kernel(
        _sc_body,
        out_type=jax.ShapeDtypeStruct((2 * N, HID_DIM), F32),
        mesh=mesh,
        compiler_params=pltpu.CompilerParams(use_tc_tiling_on_sc=False,
                                             needs_layout_passes=False),
        scratch_types=[
            pltpu.VMEM((EB, DROW), F32),
            pltpu.VMEM((EB, DROW), F32),
            pltpu.VMEM((EB, HID_DIM), F32),
            pltpu.VMEM((EPT,), F32),
            pltpu.VMEM((EB,), I32),
            pltpu.VMEM((EB,), I32),
            pltpu.VMEM((EB,), I32),
            pltpu.VMEM((EB,), I32),
            pltpu.VMEM((NS, LANES), F32),
            pltpu.VMEM((1, LANES), F32),
            pltpu.VMEM_SHARED((N, HID_DIM), F32),
            pltpu.VMEM_SHARED((NS, LANES), F32),
            pltpu.VMEM_SHARED((NS, LANES), F32),
            pltpu.SemaphoreType.DMA,
            pltpu.SemaphoreType.DMA,
            pltpu.SemaphoreType.DMA,
        ],
    )
    return f(edges_flat, u_tab, v_tab, hp_tab, zeros)


def _combine_body(h_ref, m0_ref, m1_ref, wst_ref, bs_ref, out_ref):
    x = h_ref[...]
    self_msg = jnp.dot(x, wst_ref[...], preferred_element_type=F32) + bs_ref[...]
    out_ref[...] = jnp.maximum(m0_ref[...] + m1_ref[...] + self_msg, 0.0)


def _combine(h, msgs, wst, bs):
    RB = 1000
    return pl.pallas_call(
        _combine_body,
        grid=(N // RB,),
        in_specs=[
            pl.BlockSpec((RB, IN_DIM), lambda i: (i, 0)),
            pl.BlockSpec((RB, HID_DIM), lambda i: (i, 0)),
            pl.BlockSpec((RB, HID_DIM), lambda i: (N // RB + i, 0)),
            pl.BlockSpec((IN_DIM, HID_DIM), lambda i: (0, 0)),
            pl.BlockSpec((1, HID_DIM), lambda i: (0, 0)),
        ],
        out_specs=pl.BlockSpec((RB, HID_DIM), lambda i: (i, 0)),
        out_shape=jax.ShapeDtypeStruct((N, HID_DIM), F32),
    )(h, msgs, msgs, wst, bs)


def kernel(h, node_embeddings, pos_edge_index, neg_edge_index,
           Wq_pos, bq_pos, Wk_pos, bk_pos, u_pos,
           Wq_neg, bq_neg, Wk_neg, bk_neg, u_neg,
           W_pos, b_pos, W_neg, b_neg, W_self, b_self):
    pos = pos_edge_index.astype(I32)
    neg = neg_edge_index.astype(I32)
    edges_flat = jnp.concatenate([pos.reshape(-1), neg.reshape(-1)])

    wqt_s = jnp.stack([Wq_pos.T, Wq_neg.T])
    bq_s = jnp.stack([bq_pos, bq_neg])[:, None, :]
    wkt_s = jnp.stack([Wk_pos.T, Wk_neg.T])
    bk_s = jnp.stack([bk_pos, bk_neg])[:, None, :]
    wt_s = jnp.stack([W_pos.T, W_neg.T])
    b_s = jnp.stack([b_pos, b_neg])[:, None, :]
    u_s = jnp.stack([u_pos, u_neg])[:, None, :]

    u_tab, v_tab, hp_tab = _build_tables(h, node_embeddings, wqt_s, bq_s,
                                         wkt_s, bk_s, wt_s, b_s, u_s)
    zeros = jnp.zeros((N, HID_DIM), F32)
    msgs = _sc_stage(edges_flat, u_tab, v_tab, hp_tab, zeros)
    return _combine(h, msgs, W_self.T, b_self[None, :])
